# Initial kernel scaffold; baseline (speedup 1.0000x reference)
import functools

import jax
import jax.numpy as jnp
from jax import lax
from jax.experimental import pallas as pl
from jax.experimental.pallas import tpu as pltpu
from jax.experimental.pallas import tpu_sc as plsc

N = 10000
FEAT = 128
HID = 64

NPAD = 10048
ROWS_PER_TILE = NPAD // 16
NW = 32
CHUNK = 128

_MESH = plsc.VectorSubcoreMesh(
    core_axis_name="c", subcore_axis_name="s", num_cores=2, num_subcores=16
)


def _deg_kernel(nchunks):
    def body(dst_hbm, out_hbm, idx_d, ones_v, stage_v, acc_sh, sem):
        c = lax.axis_index("c")
        s = lax.axis_index("s")
        w = c * 16 + s
        ones16 = jnp.ones((16,), jnp.float32)
        zero16 = jnp.zeros((16,), jnp.float32)

        def fill_ones(i, carry):
            ones_v[i, :] = ones16
            return carry

        lax.fori_loop(0, CHUNK, fill_ones, 0)

        def fill_zero(i, carry):
            stage_v[i, :] = zero16
            return carry

        lax.fori_loop(0, ROWS_PER_TILE, fill_zero, 0)

        pltpu.sync_copy(stage_v, acc_sh.at[pl.ds(s * ROWS_PER_TILE, ROWS_PER_TILE)])
        pltpu.sync_copy(dst_hbm.at[w], idx_d)
        plsc.subcore_barrier()

        def step(j, carry):
            pltpu.sync_copy(ones_v, acc_sh.at[idx_d.at[j]], add=True)
            return carry

        lax.fori_loop(0, nchunks, step, 0)
        plsc.subcore_barrier()
        pltpu.sync_copy(
            acc_sh.at[pl.ds(s * ROWS_PER_TILE, ROWS_PER_TILE)],
            out_hbm.at[c, pl.ds(s * ROWS_PER_TILE, ROWS_PER_TILE)],
        )

    return pl.kernel(
        body,
        out_type=jax.ShapeDtypeStruct((2, NPAD, 16), jnp.float32),
        mesh=_MESH,
        scratch_types=[
            pltpu.VMEM((nchunks, CHUNK), jnp.int32),
            pltpu.VMEM((CHUNK, 16), jnp.float32),
            pltpu.VMEM((ROWS_PER_TILE, 16), jnp.float32),
            pltpu.VMEM_SHARED((NPAD, 16), jnp.float32),
            pltpu.SemaphoreType.DMA,
        ],
    )


def _agg_kernel(nchunks, width):
    def body(h_hbm, src_hbm, dst_hbm, out_hbm,
             idx_s, idx_d, rows_v, stage_v, acc_sh, sem):
        c = lax.axis_index("c")
        s = lax.axis_index("s")
        w = c * 16 + s
        zero16 = jnp.zeros((16,), jnp.float32)

        def fill_zero(i, carry):
            for k in range(width // 16):
                stage_v[i, pl.ds(k * 16, 16)] = zero16
            return carry

        lax.fori_loop(0, ROWS_PER_TILE, fill_zero, 0)

        pltpu.sync_copy(stage_v, acc_sh.at[pl.ds(s * ROWS_PER_TILE, ROWS_PER_TILE)])
        pltpu.sync_copy(src_hbm.at[w], idx_s)
        pltpu.sync_copy(dst_hbm.at[w], idx_d)
        plsc.subcore_barrier()

        def step(j, carry):
            pltpu.async_copy(h_hbm.at[idx_s.at[j]], rows_v, sem).wait()
            pltpu.sync_copy(rows_v, acc_sh.at[idx_d.at[j]], add=True)
            return carry

        lax.fori_loop(0, nchunks, step, 0)
        plsc.subcore_barrier()
        pltpu.sync_copy(
            acc_sh.at[pl.ds(s * ROWS_PER_TILE, ROWS_PER_TILE)],
            out_hbm.at[c, pl.ds(s * ROWS_PER_TILE, ROWS_PER_TILE)],
        )

    return pl.kernel(
        body,
        out_type=jax.ShapeDtypeStruct((2, NPAD, width), jnp.float32),
        mesh=_MESH,
        scratch_types=[
            pltpu.VMEM((nchunks, CHUNK), jnp.int32),
            pltpu.VMEM((nchunks, CHUNK), jnp.int32),
            pltpu.VMEM((CHUNK, width), jnp.float32),
            pltpu.VMEM((ROWS_PER_TILE, width), jnp.float32),
            pltpu.VMEM_SHARED((NPAD, width), jnp.float32),
            pltpu.SemaphoreType.DMA,
        ],
    )


_BR = 1256


def _dinv_block(deg_ref):
    deg = deg_ref[0][:, :1] + deg_ref[1][:, :1]
    return lax.rsqrt(jnp.maximum(deg, 1.0))


def _first_mm(x, w1, degp):
    def body(x_ref, w_ref, deg_ref, o_ref):
        dinv = _dinv_block(deg_ref)
        o_ref[...] = jnp.dot(
            x_ref[...], w_ref[...], preferred_element_type=jnp.float32
        ) * dinv

    fi, fo = w1.shape
    return pl.pallas_call(
        body,
        grid=(NPAD // _BR,),
        in_specs=[
            pl.BlockSpec((_BR, fi), lambda i: (i, 0)),
            pl.BlockSpec((fi, fo), lambda i: (0, 0)),
            pl.BlockSpec((2, _BR, 16), lambda i: (0, i, 0)),
        ],
        out_specs=pl.BlockSpec((_BR, fo), lambda i: (i, 0)),
        out_shape=jax.ShapeDtypeStruct((NPAD, fo), jnp.float32),
    )(x, w1, degp)


def _mid_layer(parts, degp, b2d, wn):
    def body(p_ref, deg_ref, b_ref, w_ref, o_ref):
        dinv = _dinv_block(deg_ref)
        u = jax.nn.relu((p_ref[0] + p_ref[1]) * dinv + b_ref[0:1, :])
        o_ref[...] = jnp.dot(
            u, w_ref[...], preferred_element_type=jnp.float32
        ) * dinv

    fi, fo = wn.shape
    return pl.pallas_call(
        body,
        grid=(NPAD // _BR,),
        in_specs=[
            pl.BlockSpec((2, _BR, fi), lambda i: (0, i, 0)),
            pl.BlockSpec((2, _BR, 16), lambda i: (0, i, 0)),
            pl.BlockSpec((8, fi), lambda i: (0, 0)),
            pl.BlockSpec((fi, fo), lambda i: (0, 0)),
        ],
        out_specs=pl.BlockSpec((_BR, fo), lambda i: (i, 0)),
        out_shape=jax.ShapeDtypeStruct((NPAD, fo), jnp.float32),
    )(parts, degp, b2d, wn)


def _hidden_layer(parts, degp, b2d, wa, ws):
    def body(p_ref, deg_ref, b_ref, wa_ref, ws_ref, oa_ref, os_ref):
        dinv = _dinv_block(deg_ref)
        u = jax.nn.relu((p_ref[0] + p_ref[1]) * dinv + b_ref[0:1, :])
        oa_ref[...] = jnp.dot(
            u, wa_ref[...], preferred_element_type=jnp.float32
        ) * dinv
        os_ref[...] = jnp.dot(
            u, ws_ref[...], preferred_element_type=jnp.float32
        ) * dinv

    return pl.pallas_call(
        body,
        grid=(NPAD // _BR,),
        in_specs=[
            pl.BlockSpec((2, _BR, HID), lambda i: (0, i, 0)),
            pl.BlockSpec((2, _BR, 16), lambda i: (0, i, 0)),
            pl.BlockSpec((8, HID), lambda i: (0, 0)),
            pl.BlockSpec((HID, HID), lambda i: (0, 0)),
            pl.BlockSpec((HID, HID), lambda i: (0, 0)),
        ],
        out_specs=[
            pl.BlockSpec((_BR, HID), lambda i: (i, 0)),
            pl.BlockSpec((_BR, HID), lambda i: (i, 0)),
        ],
        out_shape=[
            jax.ShapeDtypeStruct((NPAD, HID), jnp.float32),
            jax.ShapeDtypeStruct((NPAD, HID), jnp.float32),
        ],
    )(parts, degp, b2d, wa, ws)


def _final_act(parts, degp, b2d, width):
    def body(p_ref, deg_ref, b_ref, o_ref):
        dinv = _dinv_block(deg_ref)
        o_ref[...] = jax.nn.relu((p_ref[0] + p_ref[1]) * dinv + b_ref[0:1, :])

    return pl.pallas_call(
        body,
        grid=(NPAD // _BR,),
        in_specs=[
            pl.BlockSpec((2, _BR, width), lambda i: (0, i, 0)),
            pl.BlockSpec((2, _BR, 16), lambda i: (0, i, 0)),
            pl.BlockSpec((8, width), lambda i: (0, 0)),
        ],
        out_specs=pl.BlockSpec((_BR, width), lambda i: (i, 0)),
        out_shape=jax.ShapeDtypeStruct((NPAD, width), jnp.float32),
    )(parts, degp, b2d)


_BS = 1024


def _struct_mm(s):
    def body(a_ref, b_ref, o_ref):
        o_ref[...] = lax.dot_general(
            a_ref[...], b_ref[...], (((1,), (1,)), ((), ())),
            preferred_element_type=jnp.float32,
        )

    nb = pl.cdiv(N, _BS)
    return pl.pallas_call(
        body,
        grid=(nb, nb),
        in_specs=[
            pl.BlockSpec((_BS, HID), lambda i, j: (i, 0)),
            pl.BlockSpec((_BS, HID), lambda i, j: (j, 0)),
        ],
        out_specs=pl.BlockSpec((_BS, _BS), lambda i, j: (i, j)),
        out_shape=jax.ShapeDtypeStruct((N, N), jnp.float32),
    )(s, s)


def kernel(x, edge_index, W1e, b1e, W2e, b2e, W1a, b1a, W2a, b2a, W1s, b1s):
    e = edge_index.shape[1]
    ea = e + N
    ep = ((ea + NW * CHUNK - 1) // (NW * CHUNK)) * (NW * CHUNK)
    nchunks = ep // (NW * CHUNK)

    loop = jnp.arange(N, dtype=jnp.int32)
    src = jnp.concatenate([edge_index[0].astype(jnp.int32), loop])
    dst = jnp.concatenate([edge_index[1].astype(jnp.int32), loop])
    src_p = jnp.full((ep,), N, jnp.int32).at[:ea].set(src).reshape(NW, nchunks, CHUNK)
    dst_p = jnp.full((ep,), N, jnp.int32).at[:ea].set(dst).reshape(NW, nchunks, CHUNK)

    x_pad = jnp.zeros((NPAD, FEAT), jnp.float32).at[:N].set(x)
    b1e2 = jnp.broadcast_to(b1e, (8, HID))
    b2e2 = jnp.broadcast_to(b2e, (8, HID))
    b1a2 = jnp.broadcast_to(b1a, (8, HID))
    b2a2 = jnp.broadcast_to(b2a, (8, FEAT))
    b1s2 = jnp.broadcast_to(b1s, (8, HID))

    deg = _deg_kernel(nchunks)(dst_p)

    agg64 = _agg_kernel(nchunks, HID)
    agg128 = _agg_kernel(nchunks, FEAT)

    t1 = _first_mm(x_pad, W1e, deg)
    p1 = agg64(t1, src_p, dst_p)
    t2 = _mid_layer(p1, deg, b1e2, W2e)
    p2 = agg64(t2, src_p, dst_p)
    t3, t5 = _hidden_layer(p2, deg, b2e2, W1a, W1s)
    p3 = agg64(t3, src_p, dst_p)
    t4 = _mid_layer(p3, deg, b1a2, W2a)
    p4 = agg128(t4, src_p, dst_p)
    x_hat = _final_act(p4, deg, b2a2, FEAT)
    p5 = agg64(t5, src_p, dst_p)
    s = _final_act(p5, deg, b1s2, HID)

    struct = _struct_mm(s)
    return (struct, x_hat[:N])


# speedup vs baseline: 1.0600x; 1.0600x over previous
import jax
import jax.numpy as jnp
from jax import lax
from jax.experimental import pallas as pl
from jax.experimental.pallas import tpu as pltpu
from jax.experimental.pallas import tpu_sc as plsc

N = 10000
FEAT = 128
HID = 64

NPAD = 10112
ROWS_PER_TILE = NPAD // 16
NT = 16
CHUNK = 128
BCH = 8

_MESH = plsc.VectorSubcoreMesh(
    core_axis_name="c", subcore_axis_name="s", num_cores=1, num_subcores=16
)


def _agg_kernel(nbatch):
    def body(h_hbm, src_hbm, dst_hbm, z_hbm, out_hbm,
             idx_s, idx_d, rows_v, acc_sh, sem):
        s = lax.axis_index("s")
        pltpu.sync_copy(
            z_hbm.at[pl.ds(s * ROWS_PER_TILE, ROWS_PER_TILE)],
            acc_sh.at[pl.ds(s * ROWS_PER_TILE, ROWS_PER_TILE)],
        )
        plsc.subcore_barrier()

        def outer(b, carry):
            pltpu.sync_copy(src_hbm.at[s, b], idx_s)
            pltpu.sync_copy(dst_hbm.at[s, b], idx_d)

            def step(j, carry2):
                pltpu.async_copy(h_hbm.at[idx_s.at[j]], rows_v, sem).wait()
                pltpu.sync_copy(rows_v, acc_sh.at[idx_d.at[j]], add=True)
                return carry2

            lax.fori_loop(0, BCH, step, 0)
            return carry

        lax.fori_loop(0, nbatch, outer, 0)
        plsc.subcore_barrier()
        pltpu.sync_copy(
            acc_sh.at[pl.ds(s * ROWS_PER_TILE, ROWS_PER_TILE)],
            out_hbm.at[pl.ds(s * ROWS_PER_TILE, ROWS_PER_TILE)],
        )

    return pl.kernel(
        body,
        out_type=jax.ShapeDtypeStruct((NPAD, FEAT), jnp.float32),
        mesh=_MESH,
        scratch_types=[
            pltpu.VMEM((BCH, CHUNK), jnp.int32),
            pltpu.VMEM((BCH, CHUNK), jnp.int32),
            pltpu.VMEM((CHUNK, FEAT), jnp.float32),
            pltpu.VMEM_SHARED((NPAD, FEAT), jnp.float32),
            pltpu.SemaphoreType.DMA,
        ],
    )


_BR = 1264


def _dinv_block(deg_ref):
    return lax.rsqrt(jnp.maximum(deg_ref[:, :1], 1.0))


def _first_mm(x, w1, deg):
    def body(x_ref, w_ref, deg_ref, o_ref):
        dinv = _dinv_block(deg_ref)
        t = jnp.dot(x_ref[...], w_ref[...], preferred_element_type=jnp.float32)
        o_ref[...] = jnp.concatenate(
            [t * dinv, jnp.zeros((_BR, FEAT - HID), jnp.float32)], axis=1
        )

    return pl.pallas_call(
        body,
        grid=(NPAD // _BR,),
        in_specs=[
            pl.BlockSpec((_BR, FEAT), lambda i: (i, 0)),
            pl.BlockSpec((FEAT, HID), lambda i: (0, 0)),
            pl.BlockSpec((_BR, FEAT), lambda i: (i, 0)),
        ],
        out_specs=pl.BlockSpec((_BR, FEAT), lambda i: (i, 0)),
        out_shape=jax.ShapeDtypeStruct((NPAD, FEAT), jnp.float32),
    )(x, w1, deg)


def _mid_layer(agg, deg, b2d, wn):
    fo = wn.shape[1]

    def body(p_ref, deg_ref, b_ref, w_ref, o_ref):
        dinv = _dinv_block(deg_ref)
        u = jax.nn.relu(p_ref[:, :HID] * dinv + b_ref[0:1, :])
        t = jnp.dot(u, w_ref[...], preferred_element_type=jnp.float32) * dinv
        if fo == FEAT:
            o_ref[...] = t
        else:
            o_ref[...] = jnp.concatenate(
                [t, jnp.zeros((_BR, FEAT - fo), jnp.float32)], axis=1
            )

    return pl.pallas_call(
        body,
        grid=(NPAD // _BR,),
        in_specs=[
            pl.BlockSpec((_BR, FEAT), lambda i: (i, 0)),
            pl.BlockSpec((_BR, FEAT), lambda i: (i, 0)),
            pl.BlockSpec((8, HID), lambda i: (0, 0)),
            pl.BlockSpec((HID, fo), lambda i: (0, 0)),
        ],
        out_specs=pl.BlockSpec((_BR, FEAT), lambda i: (i, 0)),
        out_shape=jax.ShapeDtypeStruct((NPAD, FEAT), jnp.float32),
    )(agg, deg, b2d, wn)


def _hidden_layer(agg, deg, b2d, wa, ws):
    def body(p_ref, deg_ref, b_ref, wa_ref, ws_ref, o_ref):
        dinv = _dinv_block(deg_ref)
        u = jax.nn.relu(p_ref[:, :HID] * dinv + b_ref[0:1, :])
        ta = jnp.dot(u, wa_ref[...], preferred_element_type=jnp.float32)
        ts = jnp.dot(u, ws_ref[...], preferred_element_type=jnp.float32)
        o_ref[...] = jnp.concatenate([ta, ts], axis=1) * dinv

    return pl.pallas_call(
        body,
        grid=(NPAD // _BR,),
        in_specs=[
            pl.BlockSpec((_BR, FEAT), lambda i: (i, 0)),
            pl.BlockSpec((_BR, FEAT), lambda i: (i, 0)),
            pl.BlockSpec((8, HID), lambda i: (0, 0)),
            pl.BlockSpec((HID, HID), lambda i: (0, 0)),
            pl.BlockSpec((HID, HID), lambda i: (0, 0)),
        ],
        out_specs=pl.BlockSpec((_BR, FEAT), lambda i: (i, 0)),
        out_shape=jax.ShapeDtypeStruct((NPAD, FEAT), jnp.float32),
    )(agg, deg, b2d, wa, ws)


def _final_act(agg, deg, b2d, lo, width):
    def body(p_ref, deg_ref, b_ref, o_ref):
        dinv = _dinv_block(deg_ref)
        o_ref[...] = jax.nn.relu(
            p_ref[:, lo:lo + width] * dinv + b_ref[0:1, :]
        )

    return pl.pallas_call(
        body,
        grid=(NPAD // _BR,),
        in_specs=[
            pl.BlockSpec((_BR, FEAT), lambda i: (i, 0)),
            pl.BlockSpec((_BR, FEAT), lambda i: (i, 0)),
            pl.BlockSpec((8, width), lambda i: (0, 0)),
        ],
        out_specs=pl.BlockSpec((_BR, width), lambda i: (i, 0)),
        out_shape=jax.ShapeDtypeStruct((NPAD, width), jnp.float32),
    )(agg, deg, b2d)


_BS = 1024


def _struct_mm(s):
    def body(a_ref, b_ref, o_ref):
        o_ref[...] = lax.dot_general(
            a_ref[...], b_ref[...], (((1,), (1,)), ((), ())),
            preferred_element_type=jnp.float32,
        )

    nb = pl.cdiv(N, _BS)
    return pl.pallas_call(
        body,
        grid=(nb, nb),
        in_specs=[
            pl.BlockSpec((_BS, HID), lambda i, j: (i, 0)),
            pl.BlockSpec((_BS, HID), lambda i, j: (j, 0)),
        ],
        out_specs=pl.BlockSpec((_BS, _BS), lambda i, j: (i, j)),
        out_shape=jax.ShapeDtypeStruct((N, N), jnp.float32),
    )(s, s)


def kernel(x, edge_index, W1e, b1e, W2e, b2e, W1a, b1a, W2a, b2a, W1s, b1s):
    e = edge_index.shape[1]
    ea = e + N
    unit = NT * BCH * CHUNK
    ep = ((ea + unit - 1) // unit) * unit
    nbatch = ep // unit

    loop = jnp.arange(N, dtype=jnp.int32)
    src = jnp.concatenate([edge_index[0].astype(jnp.int32), loop])
    dst = jnp.concatenate([edge_index[1].astype(jnp.int32), loop])
    trash = N + (jnp.arange(ep, dtype=jnp.int32) % (NPAD - N))
    src_p = trash.at[:ea].set(src).reshape(NT, nbatch, BCH, CHUNK)
    dst_p = trash.at[:ea].set(dst).reshape(NT, nbatch, BCH, CHUNK)

    x_pad = jnp.zeros((NPAD, FEAT), jnp.float32).at[:N].set(x)
    b1e2 = jnp.broadcast_to(b1e, (8, HID))
    b2e2 = jnp.broadcast_to(b2e, (8, HID))
    b1a2 = jnp.broadcast_to(b1a, (8, HID))
    b2a2 = jnp.broadcast_to(b2a, (8, FEAT))
    b1s2 = jnp.broadcast_to(b1s, (8, HID))

    z128 = jnp.zeros((NPAD, FEAT), jnp.float32)
    ones128 = jnp.ones((NPAD, FEAT), jnp.float32)
    src_z = jnp.zeros_like(src_p)
    agg0 = _agg_kernel(nbatch)
    agg = lambda h, sp, dp: agg0(h, sp, dp, z128)
    deg = agg(ones128, src_z, dst_p)

    t1 = _first_mm(x_pad, W1e, deg)
    p1 = agg(t1, src_p, dst_p)
    t2 = _mid_layer(p1, deg, b1e2, W2e)
    p2 = agg(t2, src_p, dst_p)
    t35 = _hidden_layer(p2, deg, b2e2, W1a, W1s)
    p35 = agg(t35, src_p, dst_p)
    t4 = _mid_layer(p35, deg, b1a2, W2a)
    p4 = agg(t4, src_p, dst_p)
    x_hat = _final_act(p4, deg, b2a2, 0, FEAT)
    s = _final_act(p35, deg, b1s2, HID, HID)

    struct = _struct_mm(s)
    return (struct, x_hat[:N])


# speedup vs baseline: 7.7620x; 7.3229x over previous
import jax
import jax.numpy as jnp
from jax import lax
from jax.experimental import pallas as pl
from jax.experimental.pallas import tpu as pltpu
from jax.experimental.pallas import tpu_sc as plsc

N = 10000
FEAT = 128
HID = 64

NPAD = 10112
ROWS_PER_TILE = NPAD // 16
NT = 16
CHUNK = 128
BCH = 8

_MESH = plsc.VectorSubcoreMesh(
    core_axis_name="c", subcore_axis_name="s", num_cores=1, num_subcores=16
)


def _agg_kernel(nbatch):
    def body(h_hbm, src_hbm, dst_hbm, z_hbm, out_hbm,
             idx_s, idx_d, rows_v, acc_sh, sem):
        s = lax.axis_index("s")
        pltpu.sync_copy(
            z_hbm.at[pl.ds(s * ROWS_PER_TILE, ROWS_PER_TILE)],
            acc_sh.at[pl.ds(s * ROWS_PER_TILE, ROWS_PER_TILE)],
        )
        plsc.subcore_barrier()

        def outer(b, carry):
            pltpu.sync_copy(src_hbm.at[s, b], idx_s)
            pltpu.sync_copy(dst_hbm.at[s, b], idx_d)

            def step(j, carry2):
                pltpu.async_copy(h_hbm.at[idx_s.at[j]], rows_v, sem).wait()
                pltpu.sync_copy(rows_v, acc_sh.at[idx_d.at[j]], add=True)
                return carry2

            lax.fori_loop(0, BCH, step, 0)
            return carry

        lax.fori_loop(0, nbatch, outer, 0)
        plsc.subcore_barrier()
        pltpu.sync_copy(
            acc_sh.at[pl.ds(s * ROWS_PER_TILE, ROWS_PER_TILE)],
            out_hbm.at[pl.ds(s * ROWS_PER_TILE, ROWS_PER_TILE)],
        )

    return pl.kernel(
        body,
        out_type=jax.ShapeDtypeStruct((NPAD, FEAT), jnp.float32),
        mesh=_MESH,
        scratch_types=[
            pltpu.VMEM((BCH, CHUNK), jnp.int32),
            pltpu.VMEM((BCH, CHUNK), jnp.int32),
            pltpu.VMEM((CHUNK, FEAT), jnp.float32),
            pltpu.VMEM_SHARED((NPAD, FEAT), jnp.float32),
            pltpu.SemaphoreType.DMA,
        ],
    )


_BR = 1264


def _dinv_block(deg_ref):
    return lax.rsqrt(jnp.maximum(deg_ref[:, :1], 1.0))


def _first_mm(x, w1, deg):
    def body(x_ref, w_ref, deg_ref, o_ref):
        dinv = _dinv_block(deg_ref)
        t = jnp.dot(x_ref[...], w_ref[...], preferred_element_type=jnp.float32)
        o_ref[...] = jnp.concatenate(
            [t * dinv, jnp.zeros((_BR, FEAT - HID), jnp.float32)], axis=1
        )

    return pl.pallas_call(
        body,
        grid=(NPAD // _BR,),
        in_specs=[
            pl.BlockSpec((_BR, FEAT), lambda i: (i, 0)),
            pl.BlockSpec((FEAT, HID), lambda i: (0, 0)),
            pl.BlockSpec((_BR, FEAT), lambda i: (i, 0)),
        ],
        out_specs=pl.BlockSpec((_BR, FEAT), lambda i: (i, 0)),
        out_shape=jax.ShapeDtypeStruct((NPAD, FEAT), jnp.float32),
    )(x, w1, deg)


def _mid_layer(agg, deg, b2d, wn):
    fo = wn.shape[1]

    def body(p_ref, deg_ref, b_ref, w_ref, o_ref):
        dinv = _dinv_block(deg_ref)
        u = jax.nn.relu(p_ref[:, :HID] * dinv + b_ref[0:1, :])
        t = jnp.dot(u, w_ref[...], preferred_element_type=jnp.float32) * dinv
        if fo == FEAT:
            o_ref[...] = t
        else:
            o_ref[...] = jnp.concatenate(
                [t, jnp.zeros((_BR, FEAT - fo), jnp.float32)], axis=1
            )

    return pl.pallas_call(
        body,
        grid=(NPAD // _BR,),
        in_specs=[
            pl.BlockSpec((_BR, FEAT), lambda i: (i, 0)),
            pl.BlockSpec((_BR, FEAT), lambda i: (i, 0)),
            pl.BlockSpec((8, HID), lambda i: (0, 0)),
            pl.BlockSpec((HID, fo), lambda i: (0, 0)),
        ],
        out_specs=pl.BlockSpec((_BR, FEAT), lambda i: (i, 0)),
        out_shape=jax.ShapeDtypeStruct((NPAD, FEAT), jnp.float32),
    )(agg, deg, b2d, wn)


def _hidden_layer(agg, deg, b2d, wa, ws):
    def body(p_ref, deg_ref, b_ref, wa_ref, ws_ref, o_ref):
        dinv = _dinv_block(deg_ref)
        u = jax.nn.relu(p_ref[:, :HID] * dinv + b_ref[0:1, :])
        ta = jnp.dot(u, wa_ref[...], preferred_element_type=jnp.float32)
        ts = jnp.dot(u, ws_ref[...], preferred_element_type=jnp.float32)
        o_ref[...] = jnp.concatenate([ta, ts], axis=1) * dinv

    return pl.pallas_call(
        body,
        grid=(NPAD // _BR,),
        in_specs=[
            pl.BlockSpec((_BR, FEAT), lambda i: (i, 0)),
            pl.BlockSpec((_BR, FEAT), lambda i: (i, 0)),
            pl.BlockSpec((8, HID), lambda i: (0, 0)),
            pl.BlockSpec((HID, HID), lambda i: (0, 0)),
            pl.BlockSpec((HID, HID), lambda i: (0, 0)),
        ],
        out_specs=pl.BlockSpec((_BR, FEAT), lambda i: (i, 0)),
        out_shape=jax.ShapeDtypeStruct((NPAD, FEAT), jnp.float32),
    )(agg, deg, b2d, wa, ws)


def _final_act(agg, deg, b2d, lo, width):
    def body(p_ref, deg_ref, b_ref, o_ref):
        dinv = _dinv_block(deg_ref)
        o_ref[...] = jax.nn.relu(
            p_ref[:, lo:lo + width] * dinv + b_ref[0:1, :]
        )

    return pl.pallas_call(
        body,
        grid=(NPAD // _BR,),
        in_specs=[
            pl.BlockSpec((_BR, FEAT), lambda i: (i, 0)),
            pl.BlockSpec((_BR, FEAT), lambda i: (i, 0)),
            pl.BlockSpec((8, width), lambda i: (0, 0)),
        ],
        out_specs=pl.BlockSpec((_BR, width), lambda i: (i, 0)),
        out_shape=jax.ShapeDtypeStruct((NPAD, width), jnp.float32),
    )(agg, deg, b2d)


_BS = 1024


def _struct_mm(s):
    def body(a_ref, b_ref, o_ref):
        o_ref[...] = lax.dot_general(
            a_ref[...], b_ref[...], (((1,), (1,)), ((), ())),
            preferred_element_type=jnp.float32,
        )

    nb = pl.cdiv(N, _BS)
    return pl.pallas_call(
        body,
        grid=(nb, nb),
        in_specs=[
            pl.BlockSpec((_BS, HID), lambda i, j: (i, 0)),
            pl.BlockSpec((_BS, HID), lambda i, j: (j, 0)),
        ],
        out_specs=pl.BlockSpec((_BS, _BS), lambda i, j: (i, j)),
        out_shape=jax.ShapeDtypeStruct((N, N), jnp.float32),
    )(s, s)


def kernel(x, edge_index, W1e, b1e, W2e, b2e, W1a, b1a, W2a, b2a, W1s, b1s):
    e = edge_index.shape[1]
    ea = e + N
    unit = NT * BCH * CHUNK
    ep = ((ea + unit - 1) // unit) * unit
    nbatch = ep // unit

    loop = jnp.arange(N, dtype=jnp.int32)
    src = jnp.concatenate([edge_index[0].astype(jnp.int32), loop])
    dst = jnp.concatenate([edge_index[1].astype(jnp.int32), loop])
    trash = N + (jnp.arange(ep, dtype=jnp.int32) % (NPAD - N))
    src_p = trash.at[:ea].set(src).reshape(NT, nbatch, BCH, CHUNK)
    dst_p = trash.at[:ea].set(dst).reshape(NT, nbatch, BCH, CHUNK)

    x_pad = jnp.zeros((NPAD, FEAT), jnp.float32).at[:N].set(x)
    b1e2 = jnp.broadcast_to(b1e, (8, HID))
    b2e2 = jnp.broadcast_to(b2e, (8, HID))
    b1a2 = jnp.broadcast_to(b1a, (8, HID))
    b2a2 = jnp.broadcast_to(b2a, (8, FEAT))
    b1s2 = jnp.broadcast_to(b1s, (8, HID))

    z128 = jnp.zeros((NPAD, FEAT), jnp.float32)
    ones128 = jnp.ones((NPAD, FEAT), jnp.float32)
    agg0 = _agg_kernel(nbatch)
    agg = lambda h, sp, dp: agg0(h, sp, dp, z128)
    deg = agg(ones128, src_p, dst_p)

    t1 = _first_mm(x_pad, W1e, deg)
    p1 = agg(t1, src_p, dst_p)
    t2 = _mid_layer(p1, deg, b1e2, W2e)
    p2 = agg(t2, src_p, dst_p)
    t35 = _hidden_layer(p2, deg, b2e2, W1a, W1s)
    p35 = agg(t35, src_p, dst_p)
    t4 = _mid_layer(p35, deg, b1a2, W2a)
    p4 = agg(t4, src_p, dst_p)
    x_hat = _final_act(p4, deg, b2a2, 0, FEAT)
    s = _final_act(p35, deg, b1s2, HID, HID)

    struct = _struct_mm(s)
    return (struct, x_hat[:N])


# speedup vs baseline: 11.0532x; 1.4240x over previous
import jax
import jax.numpy as jnp
from jax import lax
from jax.experimental import pallas as pl
from jax.experimental.pallas import tpu as pltpu
from jax.experimental.pallas import tpu_sc as plsc

N = 10000
FEAT = 128
HID = 64

NPAD = 10112
ROWS_PER_TILE = NPAD // 16
NT = 16
CHUNK = 128
BCH = 8

_MESH = plsc.VectorSubcoreMesh(
    core_axis_name="c", subcore_axis_name="s", num_cores=1, num_subcores=16
)


def _agg_kernel(nbatch):
    def body(h_hbm, src_hbm, dst_hbm, z_hbm, out_hbm,
             idx_s, idx_d, rows_a, rows_b, acc_sh, gs0, gs1, ss0, ss1):
        s = lax.axis_index("s")
        pltpu.sync_copy(
            z_hbm.at[pl.ds(s * ROWS_PER_TILE, ROWS_PER_TILE)],
            acc_sh.at[pl.ds(s * ROWS_PER_TILE, ROWS_PER_TILE)],
        )
        plsc.subcore_barrier()

        bufs = (rows_a, rows_b)
        gsems = (gs0, gs1)
        ssems = (ss0, ss1)

        def outer(b, carry):
            pltpu.sync_copy(src_hbm.at[s, b], idx_s)
            pltpu.sync_copy(dst_hbm.at[s, b], idx_d)

            sc_desc = [None, None]
            g_desc = [None, None]
            g_desc[0] = pltpu.async_copy(
                h_hbm.at[idx_s.at[0]], bufs[0], gsems[0]
            )
            for j in range(BCH):
                cur = j % 2
                nxt = (j + 1) % 2
                if j + 1 < BCH:
                    if sc_desc[nxt] is not None:
                        sc_desc[nxt].wait()
                        sc_desc[nxt] = None
                    g_desc[nxt] = pltpu.async_copy(
                        h_hbm.at[idx_s.at[j + 1]], bufs[nxt], gsems[nxt]
                    )
                g_desc[cur].wait()
                sc_desc[cur] = pltpu.async_copy(
                    bufs[cur], acc_sh.at[idx_d.at[j]], ssems[cur], add=True
                )
            for d in sc_desc:
                if d is not None:
                    d.wait()
            return carry

        lax.fori_loop(0, nbatch, outer, 0)
        plsc.subcore_barrier()
        pltpu.sync_copy(
            acc_sh.at[pl.ds(s * ROWS_PER_TILE, ROWS_PER_TILE)],
            out_hbm.at[pl.ds(s * ROWS_PER_TILE, ROWS_PER_TILE)],
        )

    return pl.kernel(
        body,
        out_type=jax.ShapeDtypeStruct((NPAD, FEAT), jnp.float32),
        mesh=_MESH,
        scratch_types=[
            pltpu.VMEM((BCH, CHUNK), jnp.int32),
            pltpu.VMEM((BCH, CHUNK), jnp.int32),
            pltpu.VMEM((CHUNK, FEAT), jnp.float32),
            pltpu.VMEM((CHUNK, FEAT), jnp.float32),
            pltpu.VMEM_SHARED((NPAD, FEAT), jnp.float32),
            pltpu.SemaphoreType.DMA,
            pltpu.SemaphoreType.DMA,
            pltpu.SemaphoreType.DMA,
            pltpu.SemaphoreType.DMA,
        ],
    )


_BR = 1264


def _dinv_block(deg_ref):
    return lax.rsqrt(jnp.maximum(deg_ref[:, :1], 1.0))


def _first_mm(x, w1, deg):
    def body(x_ref, w_ref, deg_ref, o_ref):
        dinv = _dinv_block(deg_ref)
        t = jnp.dot(x_ref[...], w_ref[...], preferred_element_type=jnp.float32)
        o_ref[...] = jnp.concatenate(
            [t * dinv, jnp.zeros((_BR, FEAT - HID), jnp.float32)], axis=1
        )

    return pl.pallas_call(
        body,
        grid=(NPAD // _BR,),
        in_specs=[
            pl.BlockSpec((_BR, FEAT), lambda i: (i, 0)),
            pl.BlockSpec((FEAT, HID), lambda i: (0, 0)),
            pl.BlockSpec((_BR, FEAT), lambda i: (i, 0)),
        ],
        out_specs=pl.BlockSpec((_BR, FEAT), lambda i: (i, 0)),
        out_shape=jax.ShapeDtypeStruct((NPAD, FEAT), jnp.float32),
    )(x, w1, deg)


def _mid_layer(agg, deg, b2d, wn):
    fo = wn.shape[1]

    def body(p_ref, deg_ref, b_ref, w_ref, o_ref):
        dinv = _dinv_block(deg_ref)
        u = jax.nn.relu(p_ref[:, :HID] * dinv + b_ref[0:1, :])
        t = jnp.dot(u, w_ref[...], preferred_element_type=jnp.float32) * dinv
        if fo == FEAT:
            o_ref[...] = t
        else:
            o_ref[...] = jnp.concatenate(
                [t, jnp.zeros((_BR, FEAT - fo), jnp.float32)], axis=1
            )

    return pl.pallas_call(
        body,
        grid=(NPAD // _BR,),
        in_specs=[
            pl.BlockSpec((_BR, FEAT), lambda i: (i, 0)),
            pl.BlockSpec((_BR, FEAT), lambda i: (i, 0)),
            pl.BlockSpec((8, HID), lambda i: (0, 0)),
            pl.BlockSpec((HID, fo), lambda i: (0, 0)),
        ],
        out_specs=pl.BlockSpec((_BR, FEAT), lambda i: (i, 0)),
        out_shape=jax.ShapeDtypeStruct((NPAD, FEAT), jnp.float32),
    )(agg, deg, b2d, wn)


def _hidden_layer(agg, deg, b2d, wa, ws):
    def body(p_ref, deg_ref, b_ref, wa_ref, ws_ref, o_ref):
        dinv = _dinv_block(deg_ref)
        u = jax.nn.relu(p_ref[:, :HID] * dinv + b_ref[0:1, :])
        ta = jnp.dot(u, wa_ref[...], preferred_element_type=jnp.float32)
        ts = jnp.dot(u, ws_ref[...], preferred_element_type=jnp.float32)
        o_ref[...] = jnp.concatenate([ta, ts], axis=1) * dinv

    return pl.pallas_call(
        body,
        grid=(NPAD // _BR,),
        in_specs=[
            pl.BlockSpec((_BR, FEAT), lambda i: (i, 0)),
            pl.BlockSpec((_BR, FEAT), lambda i: (i, 0)),
            pl.BlockSpec((8, HID), lambda i: (0, 0)),
            pl.BlockSpec((HID, HID), lambda i: (0, 0)),
            pl.BlockSpec((HID, HID), lambda i: (0, 0)),
        ],
        out_specs=pl.BlockSpec((_BR, FEAT), lambda i: (i, 0)),
        out_shape=jax.ShapeDtypeStruct((NPAD, FEAT), jnp.float32),
    )(agg, deg, b2d, wa, ws)


def _final_act(agg, deg, b2d, lo, width):
    def body(p_ref, deg_ref, b_ref, o_ref):
        dinv = _dinv_block(deg_ref)
        o_ref[...] = jax.nn.relu(
            p_ref[:, lo:lo + width] * dinv + b_ref[0:1, :]
        )

    return pl.pallas_call(
        body,
        grid=(NPAD // _BR,),
        in_specs=[
            pl.BlockSpec((_BR, FEAT), lambda i: (i, 0)),
            pl.BlockSpec((_BR, FEAT), lambda i: (i, 0)),
            pl.BlockSpec((8, width), lambda i: (0, 0)),
        ],
        out_specs=pl.BlockSpec((_BR, width), lambda i: (i, 0)),
        out_shape=jax.ShapeDtypeStruct((NPAD, width), jnp.float32),
    )(agg, deg, b2d)


_BS = 1024


def _struct_mm(s):
    def body(a_ref, b_ref, o_ref):
        o_ref[...] = lax.dot_general(
            a_ref[...], b_ref[...], (((1,), (1,)), ((), ())),
            preferred_element_type=jnp.float32,
        )

    nb = pl.cdiv(N, _BS)
    return pl.pallas_call(
        body,
        grid=(nb, nb),
        in_specs=[
            pl.BlockSpec((_BS, HID), lambda i, j: (i, 0)),
            pl.BlockSpec((_BS, HID), lambda i, j: (j, 0)),
        ],
        out_specs=pl.BlockSpec((_BS, _BS), lambda i, j: (i, j)),
        out_shape=jax.ShapeDtypeStruct((N, N), jnp.float32),
    )(s, s)


def kernel(x, edge_index, W1e, b1e, W2e, b2e, W1a, b1a, W2a, b2a, W1s, b1s):
    e = edge_index.shape[1]
    ea = e + N
    unit = NT * BCH * CHUNK
    ep = ((ea + unit - 1) // unit) * unit
    nbatch = ep // unit

    loop = jnp.arange(N, dtype=jnp.int32)
    src = jnp.concatenate([edge_index[0].astype(jnp.int32), loop])
    dst = jnp.concatenate([edge_index[1].astype(jnp.int32), loop])
    trash = N + (jnp.arange(ep, dtype=jnp.int32) % (NPAD - N))
    src_p = trash.at[:ea].set(src).reshape(NT, nbatch, BCH, CHUNK)
    dst_p = trash.at[:ea].set(dst).reshape(NT, nbatch, BCH, CHUNK)

    x_pad = jnp.zeros((NPAD, FEAT), jnp.float32).at[:N].set(x)
    b1e2 = jnp.broadcast_to(b1e, (8, HID))
    b2e2 = jnp.broadcast_to(b2e, (8, HID))
    b1a2 = jnp.broadcast_to(b1a, (8, HID))
    b2a2 = jnp.broadcast_to(b2a, (8, FEAT))
    b1s2 = jnp.broadcast_to(b1s, (8, HID))

    z128 = jnp.zeros((NPAD, FEAT), jnp.float32)
    ones128 = jnp.ones((NPAD, FEAT), jnp.float32)
    agg0 = _agg_kernel(nbatch)
    agg = lambda h, sp, dp: agg0(h, sp, dp, z128)
    deg = agg(ones128, src_p, dst_p)

    t1 = _first_mm(x_pad, W1e, deg)
    p1 = agg(t1, src_p, dst_p)
    t2 = _mid_layer(p1, deg, b1e2, W2e)
    p2 = agg(t2, src_p, dst_p)
    t35 = _hidden_layer(p2, deg, b2e2, W1a, W1s)
    p35 = agg(t35, src_p, dst_p)
    t4 = _mid_layer(p35, deg, b1a2, W2a)
    p4 = agg(t4, src_p, dst_p)
    x_hat = _final_act(p4, deg, b2a2, 0, FEAT)
    s = _final_act(p35, deg, b1s2, HID, HID)

    struct = _struct_mm(s)
    return (struct, x_hat[:N])
